# Initial kernel scaffold; baseline (speedup 1.0000x reference)
import jax
import jax.numpy as jnp
from jax.experimental import pallas as pl


def kernel(parent_feature, Wp, bp, We, be, Wel, bel, Wee, bee, Wne, bne, Wc, bc, Wsem, bsem, Wc2, bc2):
    raise NotImplementedError("write your pallas kernel here")


# speedup vs baseline: 29.9058x; 29.9058x over previous
import jax
import jax.numpy as jnp
from jax.experimental import pallas as pl

C = 128
H = 256
F = 256
T = 4
IT = 2
NSEM = 57

BI = 8
NBLK = C // BI
PF_BN = 4096


def _pf_body(parent_ref, wp_ref, bp_ref, out_ref):
    x = parent_ref[...]
    w = wp_ref[...]
    o = jnp.dot(x, w, preferred_element_type=jnp.float32) + bp_ref[...]
    out_ref[...] = jnp.maximum(o, 0.0)


def _b1_body(cf_ref, we_ref, be_ref, wa_ref, wb_ref, bel_ref,
             cel_ref, u_ref, v_ref):
    cf = cf_ref[...]
    cel_ref[...] = jnp.dot(cf, we_ref[...], preferred_element_type=jnp.float32) + be_ref[...]
    u_ref[...] = jnp.dot(cf, wa_ref[...], preferred_element_type=jnp.float32)
    v_ref[...] = jnp.dot(cf, wb_ref[...], preferred_element_type=jnp.float32) + bel_ref[...]


def _b2_body(u_ref, v_ref, wee_ref, bee_ref, celi_ref, celj_ref,
             eel_ref, mf_ref, coeff_ref, cnt_ref):
    u = u_ref[...]
    v = v_ref[...]
    el = jnp.maximum(u[:, None, :] + v[None, :, :], 0.0).reshape(BI * C, H)
    eel = jnp.dot(el, wee_ref[...], preferred_element_type=jnp.float32) + bee_ref[...]
    mask = (eel > 0) & (celi_ref[...] > 0) & (celj_ref[...] > 0)
    mf = mask.astype(jnp.float32)
    eel_ref[...] = eel
    mf_ref[...] = mf
    coeff_ref[...] = mf * eel
    s1 = jnp.sum(mf.reshape(BI, C, T), axis=2)
    cnt_ref[...] = jnp.sum(s1, axis=1)[:, None]


def _mp_body(cf_ref, u_ref, v_ref, mf_ref, coeff_ref, cnt_ref, cntf_ref,
             w1_ref, w2_ref, w3_ref, w4_ref, bne_ref, out_ref):
    i0 = pl.program_id(0) * BI
    cf = cf_ref[...]
    a_rows = cf_ref[pl.ds(i0, BI), :]
    a_msg = jnp.dot(a_rows, w1_ref[...], preferred_element_type=jnp.float32)
    b_msg = jnp.dot(cf, w2_ref[...], preferred_element_type=jnp.float32)
    el = jnp.maximum(u_ref[...][:, None, :] + v_ref[...][None, :, :], 0.0)
    elw = jnp.dot(el.reshape(BI * C, H), w3_ref[...],
                  preferred_element_type=jnp.float32)
    base = (elw
            + jnp.broadcast_to(a_msg[:, None, :], (BI, C, H)).reshape(BI * C, H)
            + jnp.broadcast_to(b_msg[None, :, :], (BI, C, H)).reshape(BI * C, H)
            + bne_ref[...])
    w4 = w4_ref[...]
    mf = mf_ref[...]
    coeff = coeff_ref[...]
    acc = jnp.zeros((BI * C, H), dtype=jnp.float32)
    for t in range(T):
        nef_t = jnp.maximum(base + coeff[:, t:t + 1] * w4[t:t + 1, :], 0.0)
        acc = acc + nef_t * mf[:, t:t + 1]
    sums = jnp.sum(acc.reshape(BI, C, H), axis=1)
    denom = jnp.maximum(cnt_ref[...], 1.0)
    cf_new = sums / denom
    total = jnp.sum(cntf_ref[...])
    out_ref[...] = jnp.where(total > 0.0, cf_new, a_rows)


def _final_body(c0_ref, c1_ref, c2_ref, wc_ref, bc_ref, wsem_ref, bsem_ref,
                wc2_ref, bc2_ref, feat_ref, sem_ref):
    y = (jnp.dot(c0_ref[...], wc_ref[0:H, :], preferred_element_type=jnp.float32)
         + jnp.dot(c1_ref[...], wc_ref[H:2 * H, :], preferred_element_type=jnp.float32)
         + jnp.dot(c2_ref[...], wc_ref[2 * H:3 * H, :], preferred_element_type=jnp.float32)
         + bc_ref[...])
    y = jnp.maximum(y, 0.0)
    sem_ref[...] = jnp.dot(y, wsem_ref[...], preferred_element_type=jnp.float32) + bsem_ref[...]
    feat_ref[...] = jnp.maximum(
        jnp.dot(y, wc2_ref[...], preferred_element_type=jnp.float32) + bc2_ref[...], 0.0)


def _mp_call(cf, u, v, mf, coeff, counts, wne_i, bne_i):
    w1 = wne_i[0:H]
    w2 = wne_i[H:2 * H]
    w3 = wne_i[2 * H:3 * H]
    w4 = wne_i[3 * H:3 * H + T]
    full2 = lambda shp: pl.BlockSpec(shp, lambda i: (0, 0))
    return pl.pallas_call(
        _mp_body,
        grid=(NBLK,),
        in_specs=[
            full2((C, H)),
            pl.BlockSpec((BI, H), lambda i: (i, 0)),
            full2((C, H)),
            pl.BlockSpec((BI * C, T), lambda i: (i, 0)),
            pl.BlockSpec((BI * C, T), lambda i: (i, 0)),
            pl.BlockSpec((BI, 1), lambda i: (i, 0)),
            full2((C, 1)),
            full2((H, H)), full2((H, H)), full2((H, H)),
            full2((T, H)),
            full2((1, H)),
        ],
        out_specs=pl.BlockSpec((BI, H), lambda i: (i, 0)),
        out_shape=jax.ShapeDtypeStruct((C, H), jnp.float32),
    )(cf, u, v, mf, coeff, counts, counts, w1, w2, w3, w4,
      bne_i.reshape(1, H))


def kernel(parent_feature, Wp, bp, We, be, Wel, bel, Wee, bee, Wne, bne,
           Wc, bc, Wsem, bsem, Wc2, bc2):
    n_pf = (H * C) // PF_BN
    pf = pl.pallas_call(
        _pf_body,
        grid=(n_pf,),
        in_specs=[
            pl.BlockSpec((1, F), lambda i: (0, 0)),
            pl.BlockSpec((F, PF_BN), lambda i: (0, i)),
            pl.BlockSpec((1, PF_BN), lambda i: (0, i)),
        ],
        out_specs=pl.BlockSpec((1, PF_BN), lambda i: (0, i)),
        out_shape=jax.ShapeDtypeStruct((1, H * C), jnp.float32),
    )(parent_feature, Wp, bp.reshape(1, H * C))
    cf0 = pf.reshape(C, H)

    cel, u, v = pl.pallas_call(
        _b1_body,
        out_shape=[
            jax.ShapeDtypeStruct((C, 1), jnp.float32),
            jax.ShapeDtypeStruct((C, H), jnp.float32),
            jax.ShapeDtypeStruct((C, H), jnp.float32),
        ],
    )(cf0, We, be.reshape(1, 1), Wel[0:H], Wel[H:2 * H], bel.reshape(1, H))

    celi = jnp.repeat(cel, C, axis=0)
    celj = jnp.tile(cel, (BI, 1))

    eel, mf, coeff, counts = pl.pallas_call(
        _b2_body,
        grid=(NBLK,),
        in_specs=[
            pl.BlockSpec((BI, H), lambda i: (i, 0)),
            pl.BlockSpec((C, H), lambda i: (0, 0)),
            pl.BlockSpec((H, T), lambda i: (0, 0)),
            pl.BlockSpec((1, T), lambda i: (0, 0)),
            pl.BlockSpec((BI * C, 1), lambda i: (i, 0)),
            pl.BlockSpec((BI * C, 1), lambda i: (0, 0)),
        ],
        out_specs=[
            pl.BlockSpec((BI * C, T), lambda i: (i, 0)),
            pl.BlockSpec((BI * C, T), lambda i: (i, 0)),
            pl.BlockSpec((BI * C, T), lambda i: (i, 0)),
            pl.BlockSpec((BI, 1), lambda i: (i, 0)),
        ],
        out_shape=[
            jax.ShapeDtypeStruct((C * C, T), jnp.float32),
            jax.ShapeDtypeStruct((C * C, T), jnp.float32),
            jax.ShapeDtypeStruct((C * C, T), jnp.float32),
            jax.ShapeDtypeStruct((C, 1), jnp.float32),
        ],
    )(u, v, Wee, bee.reshape(1, T), celi, celj)

    cf1 = _mp_call(cf0, u, v, mf, coeff, counts, Wne[0], bne[0])
    cf2 = _mp_call(cf1, u, v, mf, coeff, counts, Wne[1], bne[1])

    nsem_pad = 64
    wsem_p = jnp.pad(Wsem, ((0, 0), (0, nsem_pad - NSEM)))
    bsem_p = jnp.pad(bsem, (0, nsem_pad - NSEM)).reshape(1, nsem_pad)
    feats, sem = pl.pallas_call(
        _final_body,
        out_shape=[
            jax.ShapeDtypeStruct((C, F), jnp.float32),
            jax.ShapeDtypeStruct((C, nsem_pad), jnp.float32),
        ],
    )(cf0, cf1, cf2, Wc, bc.reshape(1, H), wsem_p, bsem_p, Wc2,
      bc2.reshape(1, F))

    return (feats.reshape(1, C, F),
            sem[:, :NSEM].reshape(1, C, NSEM),
            cel.reshape(1, C, 1),
            eel.reshape(1, C, C, T))


# speedup vs baseline: 33.0851x; 1.1063x over previous
import jax
import jax.numpy as jnp
from jax.experimental import pallas as pl

C = 128
H = 256
F = 256
T = 4
IT = 2
NSEM = 57

BI = 8
NBLK = C // BI
PF_BN = 4096


def _pf_body(parent_ref, wp_ref, bp_ref, out_ref):
    x = parent_ref[...]
    w = wp_ref[...]
    o = jnp.dot(x, w, preferred_element_type=jnp.float32) + bp_ref[...]
    out_ref[...] = jnp.maximum(o, 0.0)


def _b1_body(cf_ref, we_ref, be_ref, wa_ref, wb_ref, bel_ref,
             cel_ref, u_ref, v_ref):
    cf = cf_ref[...]
    cel_ref[...] = jnp.dot(cf, we_ref[...], preferred_element_type=jnp.float32) + be_ref[...]
    u_ref[...] = jnp.dot(cf, wa_ref[...], preferred_element_type=jnp.float32)
    v_ref[...] = jnp.dot(cf, wb_ref[...], preferred_element_type=jnp.float32) + bel_ref[...]


def _b2_body(u_ref, v_ref, wee_ref, bee_ref, celi_ref, celj_ref,
             eel_ref, mf_ref, coeff_ref, cnt_ref):
    u = u_ref[...]
    v = v_ref[...]
    el = jnp.maximum(u[:, None, :] + v[None, :, :], 0.0).reshape(BI * C, H)
    eel = jnp.dot(el, wee_ref[...], preferred_element_type=jnp.float32) + bee_ref[...]
    mask = (eel > 0) & (celi_ref[...] > 0) & (celj_ref[...] > 0)
    mf = mask.astype(jnp.float32)
    eel_ref[...] = eel
    mf_ref[...] = mf
    coeff_ref[...] = mf * eel
    s1 = jnp.sum(mf.reshape(BI, C, T), axis=2)
    cnt_ref[...] = jnp.sum(s1, axis=1)[:, None]


def _mp_body(cf_ref, u_ref, v_ref, mft_ref, coeff_ref, cnt_ref, cntf_ref,
             w1_ref, w2_ref, w3_ref, w4_ref, bne_ref, out_ref):
    i0 = pl.program_id(0) * BI
    cf = cf_ref[...]
    a_rows = cf_ref[pl.ds(i0, BI), :]
    a_msg = jnp.dot(a_rows, w1_ref[...], preferred_element_type=jnp.float32)
    b_msg = jnp.dot(cf, w2_ref[...],
                    preferred_element_type=jnp.float32) + bne_ref[...]
    el = jnp.maximum(u_ref[...][:, None, :] + v_ref[...][None, :, :], 0.0)
    elw = jnp.dot(el.reshape(BI * C, H), w3_ref[...],
                  preferred_element_type=jnp.float32).reshape(BI, C, H)
    base = (elw + a_msg[:, None, :] + b_msg[None, :, :]).reshape(BI * C, H)
    w4 = w4_ref[...]
    mft = mft_ref[...]
    coeff = coeff_ref[...]
    rows = jax.lax.broadcasted_iota(jnp.int32, (BI, BI * C), 0)
    cols = jax.lax.broadcasted_iota(jnp.int32, (BI, BI * C), 1)
    sel = (cols // C == rows).astype(jnp.float32)
    sums = jnp.zeros((BI, H), dtype=jnp.float32)
    for t in range(T):
        nef_t = jnp.maximum(base + coeff[:, t:t + 1] * w4[t:t + 1, :], 0.0)
        selw = sel * mft[t:t + 1, :]
        sums = sums + jnp.dot(selw, nef_t, preferred_element_type=jnp.float32)
    denom = jnp.maximum(cnt_ref[...], 1.0)
    cf_new = sums / denom
    total = jnp.sum(cntf_ref[...])
    out_ref[...] = jnp.where(total > 0.0, cf_new, a_rows)


def _final_body(c0_ref, c1_ref, c2_ref, wc_ref, bc_ref, wsem_ref, bsem_ref,
                wc2_ref, bc2_ref, feat_ref, sem_ref):
    y = (jnp.dot(c0_ref[...], wc_ref[0:H, :], preferred_element_type=jnp.float32)
         + jnp.dot(c1_ref[...], wc_ref[H:2 * H, :], preferred_element_type=jnp.float32)
         + jnp.dot(c2_ref[...], wc_ref[2 * H:3 * H, :], preferred_element_type=jnp.float32)
         + bc_ref[...])
    y = jnp.maximum(y, 0.0)
    sem_ref[...] = jnp.dot(y, wsem_ref[...], preferred_element_type=jnp.float32) + bsem_ref[...]
    feat_ref[...] = jnp.maximum(
        jnp.dot(y, wc2_ref[...], preferred_element_type=jnp.float32) + bc2_ref[...], 0.0)


def _mp_call(cf, u, v, mft, coeff, counts, wne_i, bne_i):
    w1 = wne_i[0:H]
    w2 = wne_i[H:2 * H]
    w3 = wne_i[2 * H:3 * H]
    w4 = wne_i[3 * H:3 * H + T]
    full2 = lambda shp: pl.BlockSpec(shp, lambda i: (0, 0))
    return pl.pallas_call(
        _mp_body,
        grid=(NBLK,),
        in_specs=[
            full2((C, H)),
            pl.BlockSpec((BI, H), lambda i: (i, 0)),
            full2((C, H)),
            pl.BlockSpec((T, BI * C), lambda i: (0, i)),
            pl.BlockSpec((BI * C, T), lambda i: (i, 0)),
            pl.BlockSpec((BI, 1), lambda i: (i, 0)),
            full2((C, 1)),
            full2((H, H)), full2((H, H)), full2((H, H)),
            full2((T, H)),
            full2((1, H)),
        ],
        out_specs=pl.BlockSpec((BI, H), lambda i: (i, 0)),
        out_shape=jax.ShapeDtypeStruct((C, H), jnp.float32),
    )(cf, u, v, mft, coeff, counts, counts, w1, w2, w3, w4,
      bne_i.reshape(1, H))


def kernel(parent_feature, Wp, bp, We, be, Wel, bel, Wee, bee, Wne, bne,
           Wc, bc, Wsem, bsem, Wc2, bc2):
    n_pf = (H * C) // PF_BN
    pf = pl.pallas_call(
        _pf_body,
        grid=(n_pf,),
        in_specs=[
            pl.BlockSpec((1, F), lambda i: (0, 0)),
            pl.BlockSpec((F, PF_BN), lambda i: (0, i)),
            pl.BlockSpec((1, PF_BN), lambda i: (0, i)),
        ],
        out_specs=pl.BlockSpec((1, PF_BN), lambda i: (0, i)),
        out_shape=jax.ShapeDtypeStruct((1, H * C), jnp.float32),
    )(parent_feature, Wp, bp.reshape(1, H * C))
    cf0 = pf.reshape(C, H)

    cel, u, v = pl.pallas_call(
        _b1_body,
        out_shape=[
            jax.ShapeDtypeStruct((C, 1), jnp.float32),
            jax.ShapeDtypeStruct((C, H), jnp.float32),
            jax.ShapeDtypeStruct((C, H), jnp.float32),
        ],
    )(cf0, We, be.reshape(1, 1), Wel[0:H], Wel[H:2 * H], bel.reshape(1, H))

    celi = jnp.repeat(cel, C, axis=0)
    celj = jnp.tile(cel, (BI, 1))

    eel, mf, coeff, counts = pl.pallas_call(
        _b2_body,
        grid=(NBLK,),
        in_specs=[
            pl.BlockSpec((BI, H), lambda i: (i, 0)),
            pl.BlockSpec((C, H), lambda i: (0, 0)),
            pl.BlockSpec((H, T), lambda i: (0, 0)),
            pl.BlockSpec((1, T), lambda i: (0, 0)),
            pl.BlockSpec((BI * C, 1), lambda i: (i, 0)),
            pl.BlockSpec((BI * C, 1), lambda i: (0, 0)),
        ],
        out_specs=[
            pl.BlockSpec((BI * C, T), lambda i: (i, 0)),
            pl.BlockSpec((BI * C, T), lambda i: (i, 0)),
            pl.BlockSpec((BI * C, T), lambda i: (i, 0)),
            pl.BlockSpec((BI, 1), lambda i: (i, 0)),
        ],
        out_shape=[
            jax.ShapeDtypeStruct((C * C, T), jnp.float32),
            jax.ShapeDtypeStruct((C * C, T), jnp.float32),
            jax.ShapeDtypeStruct((C * C, T), jnp.float32),
            jax.ShapeDtypeStruct((C, 1), jnp.float32),
        ],
    )(u, v, Wee, bee.reshape(1, T), celi, celj)

    mft = mf.T
    cf1 = _mp_call(cf0, u, v, mft, coeff, counts, Wne[0], bne[0])
    cf2 = _mp_call(cf1, u, v, mft, coeff, counts, Wne[1], bne[1])

    nsem_pad = 64
    wsem_p = jnp.pad(Wsem, ((0, 0), (0, nsem_pad - NSEM)))
    bsem_p = jnp.pad(bsem, (0, nsem_pad - NSEM)).reshape(1, nsem_pad)
    feats, sem = pl.pallas_call(
        _final_body,
        out_shape=[
            jax.ShapeDtypeStruct((C, F), jnp.float32),
            jax.ShapeDtypeStruct((C, nsem_pad), jnp.float32),
        ],
    )(cf0, cf1, cf2, Wc, bc.reshape(1, H), wsem_p, bsem_p, Wc2,
      bc2.reshape(1, F))

    return (feats.reshape(1, C, F),
            sem[:, :NSEM].reshape(1, C, NSEM),
            cel.reshape(1, C, 1),
            eel.reshape(1, C, C, T))


# speedup vs baseline: 35.8901x; 1.0848x over previous
import jax
import jax.numpy as jnp
from jax.experimental import pallas as pl

C = 128
H = 256
F = 256
T = 4
IT = 2
NSEM = 57

BI = 16
NBLK = C // BI
PF_BN = 4096


def _pf_body(parent_ref, wp_ref, bp_ref, out_ref):
    x = parent_ref[...]
    w = wp_ref[...]
    o = jnp.dot(x, w, preferred_element_type=jnp.float32) + bp_ref[...]
    out_ref[...] = jnp.maximum(o, 0.0)


def _b1_body(cf_ref, we_ref, be_ref, wa_ref, wb_ref, bel_ref,
             cel_ref, u_ref, v_ref):
    cf = cf_ref[...]
    cel_ref[...] = jnp.dot(cf, we_ref[...], preferred_element_type=jnp.float32) + be_ref[...]
    u_ref[...] = jnp.dot(cf, wa_ref[...], preferred_element_type=jnp.float32)
    v_ref[...] = jnp.dot(cf, wb_ref[...], preferred_element_type=jnp.float32) + bel_ref[...]


def _b2_body(u_ref, v_ref, wee_ref, bee_ref, celi_ref, celj_ref,
             eel_ref, mf_ref, coeff_ref, cnt_ref):
    u = u_ref[...]
    v = v_ref[...]
    el = jnp.maximum(u[:, None, :] + v[None, :, :], 0.0).reshape(BI * C, H)
    eel = jnp.dot(el, wee_ref[...], preferred_element_type=jnp.float32) + bee_ref[...]
    mask = (eel > 0) & (celi_ref[...] > 0) & (celj_ref[...] > 0)
    mf = mask.astype(jnp.float32)
    eel_ref[...] = eel
    mf_ref[...] = mf
    coeff_ref[...] = mf * eel
    s1 = jnp.sum(mf.reshape(BI, C, T), axis=2)
    cnt_ref[...] = jnp.sum(s1, axis=1)[:, None]


def _mp_body(cf_ref, u_ref, v_ref, mft_ref, coeff_ref, cnt_ref, cntf_ref,
             w1_ref, w2_ref, w3_ref, w4_ref, bne_ref, out_ref):
    i0 = pl.program_id(0) * BI
    cf = cf_ref[...]
    a_rows = cf_ref[pl.ds(i0, BI), :]
    a_msg = jnp.dot(a_rows, w1_ref[...], preferred_element_type=jnp.float32)
    b_msg = jnp.dot(cf, w2_ref[...],
                    preferred_element_type=jnp.float32) + bne_ref[...]
    el = jnp.maximum(u_ref[...][:, None, :] + v_ref[...][None, :, :], 0.0)
    elw = jnp.dot(el.reshape(BI * C, H), w3_ref[...],
                  preferred_element_type=jnp.float32).reshape(BI, C, H)
    base = (elw + a_msg[:, None, :] + b_msg[None, :, :]).reshape(BI * C, H)
    w4 = w4_ref[...]
    mft = mft_ref[...]
    coeff = coeff_ref[...]
    rows = jax.lax.broadcasted_iota(jnp.int32, (BI, BI * C), 0)
    cols = jax.lax.broadcasted_iota(jnp.int32, (BI, BI * C), 1)
    sel = (cols // C == rows).astype(jnp.float32)
    sums = jnp.zeros((BI, H), dtype=jnp.float32)
    for t in range(T):
        nef_t = jnp.maximum(base + coeff[:, t:t + 1] * w4[t:t + 1, :], 0.0)
        selw = sel * mft[t:t + 1, :]
        sums = sums + jnp.dot(selw, nef_t, preferred_element_type=jnp.float32)
    denom = jnp.maximum(cnt_ref[...], 1.0)
    cf_new = sums / denom
    total = jnp.sum(cntf_ref[...])
    out_ref[...] = jnp.where(total > 0.0, cf_new, a_rows)


def _final_body(c0_ref, c1_ref, c2_ref, wc_ref, bc_ref, wsem_ref, bsem_ref,
                wc2_ref, bc2_ref, feat_ref, sem_ref):
    y = (jnp.dot(c0_ref[...], wc_ref[0:H, :], preferred_element_type=jnp.float32)
         + jnp.dot(c1_ref[...], wc_ref[H:2 * H, :], preferred_element_type=jnp.float32)
         + jnp.dot(c2_ref[...], wc_ref[2 * H:3 * H, :], preferred_element_type=jnp.float32)
         + bc_ref[...])
    y = jnp.maximum(y, 0.0)
    sem_ref[...] = jnp.dot(y, wsem_ref[...], preferred_element_type=jnp.float32) + bsem_ref[...]
    feat_ref[...] = jnp.maximum(
        jnp.dot(y, wc2_ref[...], preferred_element_type=jnp.float32) + bc2_ref[...], 0.0)


def _mp_call(cf, u, v, mft, coeff, counts, wne_i, bne_i):
    w1 = wne_i[0:H]
    w2 = wne_i[H:2 * H]
    w3 = wne_i[2 * H:3 * H]
    w4 = wne_i[3 * H:3 * H + T]
    full2 = lambda shp: pl.BlockSpec(shp, lambda i: (0, 0))
    return pl.pallas_call(
        _mp_body,
        grid=(NBLK,),
        in_specs=[
            full2((C, H)),
            pl.BlockSpec((BI, H), lambda i: (i, 0)),
            full2((C, H)),
            pl.BlockSpec((T, BI * C), lambda i: (0, i)),
            pl.BlockSpec((BI * C, T), lambda i: (i, 0)),
            pl.BlockSpec((BI, 1), lambda i: (i, 0)),
            full2((C, 1)),
            full2((H, H)), full2((H, H)), full2((H, H)),
            full2((T, H)),
            full2((1, H)),
        ],
        out_specs=pl.BlockSpec((BI, H), lambda i: (i, 0)),
        out_shape=jax.ShapeDtypeStruct((C, H), jnp.float32),
    )(cf, u, v, mft, coeff, counts, counts, w1, w2, w3, w4,
      bne_i.reshape(1, H))


def kernel(parent_feature, Wp, bp, We, be, Wel, bel, Wee, bee, Wne, bne,
           Wc, bc, Wsem, bsem, Wc2, bc2):
    n_pf = (H * C) // PF_BN
    pf = pl.pallas_call(
        _pf_body,
        grid=(n_pf,),
        in_specs=[
            pl.BlockSpec((1, F), lambda i: (0, 0)),
            pl.BlockSpec((F, PF_BN), lambda i: (0, i)),
            pl.BlockSpec((1, PF_BN), lambda i: (0, i)),
        ],
        out_specs=pl.BlockSpec((1, PF_BN), lambda i: (0, i)),
        out_shape=jax.ShapeDtypeStruct((1, H * C), jnp.float32),
    )(parent_feature, Wp, bp.reshape(1, H * C))
    cf0 = pf.reshape(C, H)

    cel, u, v = pl.pallas_call(
        _b1_body,
        out_shape=[
            jax.ShapeDtypeStruct((C, 1), jnp.float32),
            jax.ShapeDtypeStruct((C, H), jnp.float32),
            jax.ShapeDtypeStruct((C, H), jnp.float32),
        ],
    )(cf0, We, be.reshape(1, 1), Wel[0:H], Wel[H:2 * H], bel.reshape(1, H))

    celi = jnp.repeat(cel, C, axis=0)
    celj = jnp.tile(cel, (BI, 1))

    eel, mf, coeff, counts = pl.pallas_call(
        _b2_body,
        grid=(NBLK,),
        in_specs=[
            pl.BlockSpec((BI, H), lambda i: (i, 0)),
            pl.BlockSpec((C, H), lambda i: (0, 0)),
            pl.BlockSpec((H, T), lambda i: (0, 0)),
            pl.BlockSpec((1, T), lambda i: (0, 0)),
            pl.BlockSpec((BI * C, 1), lambda i: (i, 0)),
            pl.BlockSpec((BI * C, 1), lambda i: (0, 0)),
        ],
        out_specs=[
            pl.BlockSpec((BI * C, T), lambda i: (i, 0)),
            pl.BlockSpec((BI * C, T), lambda i: (i, 0)),
            pl.BlockSpec((BI * C, T), lambda i: (i, 0)),
            pl.BlockSpec((BI, 1), lambda i: (i, 0)),
        ],
        out_shape=[
            jax.ShapeDtypeStruct((C * C, T), jnp.float32),
            jax.ShapeDtypeStruct((C * C, T), jnp.float32),
            jax.ShapeDtypeStruct((C * C, T), jnp.float32),
            jax.ShapeDtypeStruct((C, 1), jnp.float32),
        ],
    )(u, v, Wee, bee.reshape(1, T), celi, celj)

    mft = mf.T
    cf1 = _mp_call(cf0, u, v, mft, coeff, counts, Wne[0], bne[0])
    cf2 = _mp_call(cf1, u, v, mft, coeff, counts, Wne[1], bne[1])

    nsem_pad = 64
    wsem_p = jnp.pad(Wsem, ((0, 0), (0, nsem_pad - NSEM)))
    bsem_p = jnp.pad(bsem, (0, nsem_pad - NSEM)).reshape(1, nsem_pad)
    feats, sem = pl.pallas_call(
        _final_body,
        out_shape=[
            jax.ShapeDtypeStruct((C, F), jnp.float32),
            jax.ShapeDtypeStruct((C, nsem_pad), jnp.float32),
        ],
    )(cf0, cf1, cf2, Wc, bc.reshape(1, H), wsem_p, bsem_p, Wc2,
      bc2.reshape(1, F))

    return (feats.reshape(1, C, F),
            sem[:, :NSEM].reshape(1, C, NSEM),
            cel.reshape(1, C, 1),
            eel.reshape(1, C, C, T))


# speedup vs baseline: 36.2127x; 1.0090x over previous
import jax
import jax.numpy as jnp
from jax.experimental import pallas as pl

C = 128
H = 256
F = 256
T = 4
IT = 2
NSEM = 57

BI = 32
NBLK = C // BI
PF_BN = 8192


def _pf_body(parent_ref, wp_ref, bp_ref, out_ref):
    x = parent_ref[...]
    w = wp_ref[...]
    o = jnp.dot(x, w, preferred_element_type=jnp.float32) + bp_ref[...]
    out_ref[...] = jnp.maximum(o, 0.0)


def _b1_body(cf_ref, we_ref, be_ref, wa_ref, wb_ref, bel_ref,
             cel_ref, u_ref, v_ref):
    cf = cf_ref[...]
    cel_ref[...] = jnp.dot(cf, we_ref[...], preferred_element_type=jnp.float32) + be_ref[...]
    u_ref[...] = jnp.dot(cf, wa_ref[...], preferred_element_type=jnp.float32)
    v_ref[...] = jnp.dot(cf, wb_ref[...], preferred_element_type=jnp.float32) + bel_ref[...]


def _b2_body(u_ref, v_ref, wee_ref, bee_ref, celi_ref, celj_ref,
             eel_ref, mf_ref, coeff_ref, cnt_ref):
    u = u_ref[...]
    v = v_ref[...]
    el = jnp.maximum(u[:, None, :] + v[None, :, :], 0.0).reshape(BI * C, H)
    eel = jnp.dot(el, wee_ref[...], preferred_element_type=jnp.float32) + bee_ref[...]
    mask = (eel > 0) & (celi_ref[...] > 0) & (celj_ref[...] > 0)
    mf = mask.astype(jnp.float32)
    eel_ref[...] = eel
    mf_ref[...] = mf
    coeff_ref[...] = mf * eel
    s1 = jnp.sum(mf.reshape(BI, C, T), axis=2)
    cnt_ref[...] = jnp.sum(s1, axis=1)[:, None]


def _mp_body(cf_ref, u_ref, v_ref, mft_ref, coeff_ref, cnt_ref, cntf_ref,
             w1_ref, w2_ref, w3_ref, w4_ref, bne_ref, out_ref):
    i0 = pl.program_id(0) * BI
    cf = cf_ref[...]
    a_rows = cf_ref[pl.ds(i0, BI), :]
    a_msg = jnp.dot(a_rows, w1_ref[...], preferred_element_type=jnp.float32)
    b_msg = jnp.dot(cf, w2_ref[...],
                    preferred_element_type=jnp.float32) + bne_ref[...]
    el = jnp.maximum(u_ref[...][:, None, :] + v_ref[...][None, :, :], 0.0)
    elw = jnp.dot(el.reshape(BI * C, H), w3_ref[...],
                  preferred_element_type=jnp.float32).reshape(BI, C, H)
    base = (elw + a_msg[:, None, :] + b_msg[None, :, :]).reshape(BI * C, H)
    w4 = w4_ref[...]
    mft = mft_ref[...]
    coeff = coeff_ref[...]
    rows = jax.lax.broadcasted_iota(jnp.int32, (BI, BI * C), 0)
    cols = jax.lax.broadcasted_iota(jnp.int32, (BI, BI * C), 1)
    sel = (cols // C == rows).astype(jnp.float32)
    sums = jnp.zeros((BI, H), dtype=jnp.float32)
    for t in range(T):
        nef_t = jnp.maximum(base + coeff[:, t:t + 1] * w4[t:t + 1, :], 0.0)
        selw = sel * mft[t:t + 1, :]
        sums = sums + jnp.dot(selw, nef_t, preferred_element_type=jnp.float32)
    denom = jnp.maximum(cnt_ref[...], 1.0)
    cf_new = sums / denom
    total = jnp.sum(cntf_ref[...])
    out_ref[...] = jnp.where(total > 0.0, cf_new, a_rows)


def _final_body(c0_ref, c1_ref, c2_ref, wc_ref, bc_ref, wsem_ref, bsem_ref,
                wc2_ref, bc2_ref, feat_ref, sem_ref):
    y = (jnp.dot(c0_ref[...], wc_ref[0:H, :], preferred_element_type=jnp.float32)
         + jnp.dot(c1_ref[...], wc_ref[H:2 * H, :], preferred_element_type=jnp.float32)
         + jnp.dot(c2_ref[...], wc_ref[2 * H:3 * H, :], preferred_element_type=jnp.float32)
         + bc_ref[...])
    y = jnp.maximum(y, 0.0)
    sem_ref[...] = jnp.dot(y, wsem_ref[...], preferred_element_type=jnp.float32) + bsem_ref[...]
    feat_ref[...] = jnp.maximum(
        jnp.dot(y, wc2_ref[...], preferred_element_type=jnp.float32) + bc2_ref[...], 0.0)


def _mp_call(cf, u, v, mft, coeff, counts, wne_i, bne_i):
    w1 = wne_i[0:H]
    w2 = wne_i[H:2 * H]
    w3 = wne_i[2 * H:3 * H]
    w4 = wne_i[3 * H:3 * H + T]
    full2 = lambda shp: pl.BlockSpec(shp, lambda i: (0, 0))
    return pl.pallas_call(
        _mp_body,
        grid=(NBLK,),
        in_specs=[
            full2((C, H)),
            pl.BlockSpec((BI, H), lambda i: (i, 0)),
            full2((C, H)),
            pl.BlockSpec((T, BI * C), lambda i: (0, i)),
            pl.BlockSpec((BI * C, T), lambda i: (i, 0)),
            pl.BlockSpec((BI, 1), lambda i: (i, 0)),
            full2((C, 1)),
            full2((H, H)), full2((H, H)), full2((H, H)),
            full2((T, H)),
            full2((1, H)),
        ],
        out_specs=pl.BlockSpec((BI, H), lambda i: (i, 0)),
        out_shape=jax.ShapeDtypeStruct((C, H), jnp.float32),
    )(cf, u, v, mft, coeff, counts, counts, w1, w2, w3, w4,
      bne_i.reshape(1, H))


def kernel(parent_feature, Wp, bp, We, be, Wel, bel, Wee, bee, Wne, bne,
           Wc, bc, Wsem, bsem, Wc2, bc2):
    n_pf = (H * C) // PF_BN
    pf = pl.pallas_call(
        _pf_body,
        grid=(n_pf,),
        in_specs=[
            pl.BlockSpec((1, F), lambda i: (0, 0)),
            pl.BlockSpec((F, PF_BN), lambda i: (0, i)),
            pl.BlockSpec((1, PF_BN), lambda i: (0, i)),
        ],
        out_specs=pl.BlockSpec((1, PF_BN), lambda i: (0, i)),
        out_shape=jax.ShapeDtypeStruct((1, H * C), jnp.float32),
    )(parent_feature, Wp, bp.reshape(1, H * C))
    cf0 = pf.reshape(C, H)

    cel, u, v = pl.pallas_call(
        _b1_body,
        out_shape=[
            jax.ShapeDtypeStruct((C, 1), jnp.float32),
            jax.ShapeDtypeStruct((C, H), jnp.float32),
            jax.ShapeDtypeStruct((C, H), jnp.float32),
        ],
    )(cf0, We, be.reshape(1, 1), Wel[0:H], Wel[H:2 * H], bel.reshape(1, H))

    celi = jnp.repeat(cel, C, axis=0)
    celj = jnp.tile(cel, (BI, 1))

    eel, mf, coeff, counts = pl.pallas_call(
        _b2_body,
        grid=(NBLK,),
        in_specs=[
            pl.BlockSpec((BI, H), lambda i: (i, 0)),
            pl.BlockSpec((C, H), lambda i: (0, 0)),
            pl.BlockSpec((H, T), lambda i: (0, 0)),
            pl.BlockSpec((1, T), lambda i: (0, 0)),
            pl.BlockSpec((BI * C, 1), lambda i: (i, 0)),
            pl.BlockSpec((BI * C, 1), lambda i: (0, 0)),
        ],
        out_specs=[
            pl.BlockSpec((BI * C, T), lambda i: (i, 0)),
            pl.BlockSpec((BI * C, T), lambda i: (i, 0)),
            pl.BlockSpec((BI * C, T), lambda i: (i, 0)),
            pl.BlockSpec((BI, 1), lambda i: (i, 0)),
        ],
        out_shape=[
            jax.ShapeDtypeStruct((C * C, T), jnp.float32),
            jax.ShapeDtypeStruct((C * C, T), jnp.float32),
            jax.ShapeDtypeStruct((C * C, T), jnp.float32),
            jax.ShapeDtypeStruct((C, 1), jnp.float32),
        ],
    )(u, v, Wee, bee.reshape(1, T), celi, celj)

    mft = mf.T
    cf1 = _mp_call(cf0, u, v, mft, coeff, counts, Wne[0], bne[0])
    cf2 = _mp_call(cf1, u, v, mft, coeff, counts, Wne[1], bne[1])

    nsem_pad = 64
    wsem_p = jnp.pad(Wsem, ((0, 0), (0, nsem_pad - NSEM)))
    bsem_p = jnp.pad(bsem, (0, nsem_pad - NSEM)).reshape(1, nsem_pad)
    feats, sem = pl.pallas_call(
        _final_body,
        out_shape=[
            jax.ShapeDtypeStruct((C, F), jnp.float32),
            jax.ShapeDtypeStruct((C, nsem_pad), jnp.float32),
        ],
    )(cf0, cf1, cf2, Wc, bc.reshape(1, H), wsem_p, bsem_p, Wc2,
      bc2.reshape(1, F))

    return (feats.reshape(1, C, F),
            sem[:, :NSEM].reshape(1, C, NSEM),
            cel.reshape(1, C, 1),
            eel.reshape(1, C, C, T))


# speedup vs baseline: 39.9302x; 1.1027x over previous
import jax
import jax.numpy as jnp
from jax.experimental import pallas as pl
from jax.experimental.pallas import tpu as pltpu

C = 128
H = 256
F = 256
T = 4
IT = 2
NSEM = 57
NSEM_PAD = 64

BI = 32
NBLK = C // BI
PF_BN = BI * H
BIG = 1e30

_S_A = 0
_S_B1 = _S_A + NBLK
_S_B2 = _S_B1 + 1
_S_MP0 = _S_B2 + NBLK
_S_MP1 = _S_MP0 + NBLK
_S_D = _S_MP1 + NBLK
_NSTEPS = _S_D + 1


def _dot(a, b):
    return jnp.dot(a, b, preferred_element_type=jnp.float32)


def _mp_stage(i, cf_s, u_s, v_s, coeff_s, mfm1_s, cnt_s,
              w1_ref, w2_ref, w3_ref, w4big_ref, bne_ref, out_s):
    i0 = i * BI
    cf = cf_s[...]
    a_rows = cf_s[pl.ds(i0, BI), :]
    a_msg = _dot(a_rows, w1_ref[...])
    b_msg = _dot(cf, w2_ref[...]) + bne_ref[...]
    el = jnp.maximum(u_s[pl.ds(i0, BI), :][:, None, :] + v_s[...][None, :, :],
                     0.0)
    elw = _dot(el.reshape(BI * C, H), w3_ref[...]).reshape(BI, C, H)
    base = (elw + a_msg[:, None, :] + b_msg[None, :, :]).reshape(BI * C, H)
    coeff = coeff_s[pl.ds(i0 * C, BI * C), :]
    mfm1 = mfm1_s[pl.ds(i0 * C, BI * C), :]
    rows = jax.lax.broadcasted_iota(jnp.int32, (BI, BI * C), 0)
    cols = jax.lax.broadcasted_iota(jnp.int32, (BI, BI * C), 1)
    sel = (cols // C == rows).astype(jnp.float32)
    sums = jnp.zeros((BI, H), dtype=jnp.float32)
    for t in range(T):
        lhs = jnp.concatenate([coeff[:, t:t + 1], mfm1[:, t:t + 1]], axis=1)
        z_t = _dot(lhs, w4big_ref[pl.ds(2 * t, 2), :])
        nef_t = jnp.maximum(base + z_t, 0.0)
        sums = sums + _dot(sel, nef_t)
    denom = jnp.maximum(cnt_s[pl.ds(i0, BI), :], 1.0)
    cf_new = sums / denom
    total = jnp.sum(cnt_s[...])
    out_s[pl.ds(i0, BI), :] = jnp.where(total > 0.0, cf_new, a_rows)


def _body(parent_ref, wp_ref, bp_ref, we_ref, be_ref, wela_ref, welb_ref,
          bel_ref, wee_ref, bee_ref,
          w1a_ref, w2a_ref, w3a_ref, w4biga_ref, bnea_ref,
          w1b_ref, w2b_ref, w3b_ref, w4bigb_ref, bneb_ref,
          wc_ref, bc_ref, wsem_ref, bsem_ref, wc2_ref, bc2_ref,
          cel_ref, eel_ref, feat_ref, sem_ref,
          cf0_s, u_s, v_s, cel_s, coeff_s, mfm1_s, cnt_s, cf1_s, cf2_s):
    s = pl.program_id(0)

    @pl.when(s < _S_B1)
    def _stage_a():
        o = jnp.maximum(_dot(parent_ref[...], wp_ref[...]) + bp_ref[...], 0.0)
        cf0_s[pl.ds(s * BI, BI), :] = o.reshape(BI, H)

    @pl.when(s == _S_B1)
    def _stage_b1():
        cf = cf0_s[...]
        cel = _dot(cf, we_ref[...]) + be_ref[...]
        cel_s[...] = cel
        cel_ref[...] = cel
        u_s[...] = _dot(cf, wela_ref[...])
        v_s[...] = _dot(cf, welb_ref[...]) + bel_ref[...]

    @pl.when((s >= _S_B2) & (s < _S_MP0))
    def _stage_b2():
        i = s - _S_B2
        i0 = i * BI
        u_b = u_s[pl.ds(i0, BI), :]
        el = jnp.maximum(u_b[:, None, :] + v_s[...][None, :, :],
                         0.0).reshape(BI * C, H)
        eel = _dot(el, wee_ref[...]) + bee_ref[...]
        eel_ref[...] = eel
        cel = cel_s[...]
        celj = jnp.broadcast_to(cel.reshape(1, C, 1),
                                (BI, C, 1)).reshape(BI * C, 1)
        celi = jnp.broadcast_to(cel_s[pl.ds(i0, BI), :][:, None, :],
                                (BI, C, 1)).reshape(BI * C, 1)
        mask = (eel > 0) & (celi > 0) & (celj > 0)
        mf = mask.astype(jnp.float32)
        coeff_s[pl.ds(i0 * C, BI * C), :] = mf * eel
        mfm1_s[pl.ds(i0 * C, BI * C), :] = (mf - 1.0) * BIG
        s1 = jnp.sum(mf.reshape(BI, C, T), axis=2)
        cnt_s[pl.ds(i0, BI), :] = jnp.sum(s1, axis=1)[:, None]

    @pl.when((s >= _S_MP0) & (s < _S_MP1))
    def _stage_mp0():
        _mp_stage(s - _S_MP0, cf0_s, u_s, v_s, coeff_s, mfm1_s, cnt_s,
                  w1a_ref, w2a_ref, w3a_ref, w4biga_ref, bnea_ref, cf1_s)

    @pl.when((s >= _S_MP1) & (s < _S_D))
    def _stage_mp1():
        _mp_stage(s - _S_MP1, cf1_s, u_s, v_s, coeff_s, mfm1_s, cnt_s,
                  w1b_ref, w2b_ref, w3b_ref, w4bigb_ref, bneb_ref, cf2_s)

    @pl.when(s == _S_D)
    def _stage_d():
        y = (_dot(cf0_s[...], wc_ref[0:H, :])
             + _dot(cf1_s[...], wc_ref[H:2 * H, :])
             + _dot(cf2_s[...], wc_ref[2 * H:3 * H, :])
             + bc_ref[...])
        y = jnp.maximum(y, 0.0)
        sem_ref[...] = _dot(y, wsem_ref[...]) + bsem_ref[...]
        feat_ref[...] = jnp.maximum(_dot(y, wc2_ref[...]) + bc2_ref[...], 0.0)


def kernel(parent_feature, Wp, bp, We, be, Wel, bel, Wee, bee, Wne, bne,
           Wc, bc, Wsem, bsem, Wc2, bc2):
    wsem_p = jnp.pad(Wsem, ((0, 0), (0, NSEM_PAD - NSEM)))
    bsem_p = jnp.pad(bsem, (0, NSEM_PAD - NSEM)).reshape(1, NSEM_PAD)
    big_row = jnp.full((1, H), BIG, dtype=jnp.float32)

    def wne_slices(i):
        w4big = jnp.concatenate(
            [jnp.stack([Wne[i, 3 * H + t], big_row[0]]) for t in range(T)], 0)
        return (Wne[i, 0:H], Wne[i, H:2 * H], Wne[i, 2 * H:3 * H],
                w4big, bne[i].reshape(1, H))

    w1a, w2a, w3a, w4biga, bnea = wne_slices(0)
    w1b, w2b, w3b, w4bigb, bneb = wne_slices(1)

    full = lambda shp: pl.BlockSpec(shp, lambda s: tuple(0 for _ in shp))
    wp_spec = pl.BlockSpec((F, PF_BN), lambda s: (0, jnp.minimum(s, NBLK - 1)))
    bp_spec = pl.BlockSpec((1, PF_BN), lambda s: (0, jnp.minimum(s, NBLK - 1)))
    eel_spec = pl.BlockSpec(
        (BI * C, T), lambda s: (jnp.clip(s - _S_B2, 0, NBLK - 1), 0))

    cel, eel, feats, sem = pl.pallas_call(
        _body,
        grid=(_NSTEPS,),
        in_specs=[
            full((1, F)), wp_spec, bp_spec,
            full((H, 1)), full((1, 1)), full((H, H)), full((H, H)),
            full((1, H)), full((H, T)), full((1, T)),
            full((H, H)), full((H, H)), full((H, H)), full((2 * T, H)),
            full((1, H)),
            full((H, H)), full((H, H)), full((H, H)), full((2 * T, H)),
            full((1, H)),
            full((3 * H, H)), full((1, H)), full((H, NSEM_PAD)),
            full((1, NSEM_PAD)), full((H, F)), full((1, F)),
        ],
        out_specs=[
            full((C, 1)), eel_spec, full((C, F)), full((C, NSEM_PAD)),
        ],
        out_shape=[
            jax.ShapeDtypeStruct((C, 1), jnp.float32),
            jax.ShapeDtypeStruct((C * C, T), jnp.float32),
            jax.ShapeDtypeStruct((C, F), jnp.float32),
            jax.ShapeDtypeStruct((C, NSEM_PAD), jnp.float32),
        ],
        scratch_shapes=[
            pltpu.VMEM((C, H), jnp.float32),
            pltpu.VMEM((C, H), jnp.float32),
            pltpu.VMEM((C, H), jnp.float32),
            pltpu.VMEM((C, 1), jnp.float32),
            pltpu.VMEM((C * C, T), jnp.float32),
            pltpu.VMEM((C * C, T), jnp.float32),
            pltpu.VMEM((C, 1), jnp.float32),
            pltpu.VMEM((C, H), jnp.float32),
            pltpu.VMEM((C, H), jnp.float32),
        ],
    )(parent_feature, Wp, bp.reshape(1, H * C), We, be.reshape(1, 1),
      Wel[0:H], Wel[H:2 * H], bel.reshape(1, H), Wee, bee.reshape(1, T),
      w1a, w2a, w3a, w4biga, bnea, w1b, w2b, w3b, w4bigb, bneb,
      Wc, bc.reshape(1, H), wsem_p, bsem_p, Wc2, bc2.reshape(1, F))

    return (feats.reshape(1, C, F),
            sem[:, :NSEM].reshape(1, C, NSEM),
            cel.reshape(1, C, 1),
            eel.reshape(1, C, C, T))


# speedup vs baseline: 43.9757x; 1.1013x over previous
import jax
import jax.numpy as jnp
from jax.experimental import pallas as pl
from jax.experimental.pallas import tpu as pltpu

C = 128
H = 256
F = 256
T = 4
IT = 2
NSEM = 57
NSEM_PAD = 64

BI = 32
NBLK = C // BI
PF_BN = 2048
PF_ROWS = PF_BN // H
N_A = (H * C) // PF_BN
BIG = 1e30

_S_A = 0
_S_B1 = _S_A + N_A
_S_B2 = _S_B1 + 1
_S_MP0 = _S_B2 + NBLK
_S_MP1 = _S_MP0 + NBLK
_S_D = _S_MP1 + NBLK
_NSTEPS = _S_D + 1


def _dot(a, b):
    return jnp.dot(a, b, preferred_element_type=jnp.float32)


def _mp_stage(i, cf_s, el_s, coeff_s, mfm1_s, cnt_s,
              w1_ref, w2_ref, w3_ref, w4big_ref, bne_ref, out_s):
    i0 = i * BI
    cf = cf_s[...]
    a_rows = cf_s[pl.ds(i0, BI), :]
    a_msg = _dot(a_rows, w1_ref[...])
    b_msg = _dot(cf, w2_ref[...]) + bne_ref[...]
    el = el_s[pl.ds(i0 * C, BI * C), :]
    elw = _dot(el, w3_ref[...]).reshape(BI, C, H)
    base = (elw + a_msg[:, None, :] + b_msg[None, :, :]).reshape(BI * C, H)
    coeff = coeff_s[pl.ds(i0 * C, BI * C), :]
    mfm1 = mfm1_s[pl.ds(i0 * C, BI * C), :]
    nt = jnp.zeros((BI * C, H), dtype=jnp.float32)
    for t in range(T):
        lhs = jnp.concatenate([coeff[:, t:t + 1], mfm1[:, t:t + 1]], axis=1)
        z_t = _dot(lhs, w4big_ref[pl.ds(2 * t, 2), :])
        nt = nt + jnp.maximum(base + z_t, 0.0)
    rows = jax.lax.broadcasted_iota(jnp.int32, (BI, BI * C), 0)
    cols = jax.lax.broadcasted_iota(jnp.int32, (BI, BI * C), 1)
    sel = (cols // C == rows).astype(jnp.float32)
    sums = _dot(sel, nt)
    denom = jnp.maximum(cnt_s[pl.ds(i0, BI), :], 1.0)
    cf_new = sums / denom
    total = jnp.sum(cnt_s[...])
    out_s[pl.ds(i0, BI), :] = jnp.where(total > 0.0, cf_new, a_rows)


def _body(parent_ref, wp_ref, bp_ref, we_ref, be_ref, wela_ref, welb_ref,
          bel_ref, wee_ref, bee_ref,
          w1a_ref, w2a_ref, w3a_ref, w4biga_ref, bnea_ref,
          w1b_ref, w2b_ref, w3b_ref, w4bigb_ref, bneb_ref,
          wc_ref, bc_ref, wsem_ref, bsem_ref, wc2_ref, bc2_ref,
          cel_ref, eel_ref, feat_ref, sem_ref,
          cf0_s, u_s, v_s, cel_s, el_s, coeff_s, mfm1_s, cnt_s, cf1_s, cf2_s):
    s = pl.program_id(0)

    @pl.when(s < _S_B1)
    def _stage_a():
        o = jnp.maximum(_dot(parent_ref[...], wp_ref[...]) + bp_ref[...], 0.0)
        cf0_s[pl.ds(s * PF_ROWS, PF_ROWS), :] = o.reshape(PF_ROWS, H)

    @pl.when(s == _S_B1)
    def _stage_b1():
        cf = cf0_s[...]
        cel = _dot(cf, we_ref[...]) + be_ref[...]
        cel_s[...] = cel
        cel_ref[...] = cel
        u_s[...] = _dot(cf, wela_ref[...])
        v_s[...] = _dot(cf, welb_ref[...]) + bel_ref[...]

    @pl.when((s >= _S_B2) & (s < _S_MP0))
    def _stage_b2():
        i = s - _S_B2
        i0 = i * BI
        u_b = u_s[pl.ds(i0, BI), :]
        el = jnp.maximum(u_b[:, None, :] + v_s[...][None, :, :],
                         0.0).reshape(BI * C, H)
        el_s[pl.ds(i0 * C, BI * C), :] = el
        eel = _dot(el, wee_ref[...]) + bee_ref[...]
        eel_ref[...] = eel
        cel = cel_s[...]
        celj = jnp.broadcast_to(cel.reshape(1, C, 1),
                                (BI, C, 1)).reshape(BI * C, 1)
        celi = jnp.broadcast_to(cel_s[pl.ds(i0, BI), :][:, None, :],
                                (BI, C, 1)).reshape(BI * C, 1)
        mask = (eel > 0) & (celi > 0) & (celj > 0)
        mf = mask.astype(jnp.float32)
        coeff_s[pl.ds(i0 * C, BI * C), :] = mf * eel
        mfm1_s[pl.ds(i0 * C, BI * C), :] = (mf - 1.0) * BIG
        s1 = jnp.sum(mf.reshape(BI, C, T), axis=2)
        cnt_s[pl.ds(i0, BI), :] = jnp.sum(s1, axis=1)[:, None]

    @pl.when((s >= _S_MP0) & (s < _S_MP1))
    def _stage_mp0():
        _mp_stage(s - _S_MP0, cf0_s, el_s, coeff_s, mfm1_s, cnt_s,
                  w1a_ref, w2a_ref, w3a_ref, w4biga_ref, bnea_ref, cf1_s)

    @pl.when((s >= _S_MP1) & (s < _S_D))
    def _stage_mp1():
        _mp_stage(s - _S_MP1, cf1_s, el_s, coeff_s, mfm1_s, cnt_s,
                  w1b_ref, w2b_ref, w3b_ref, w4bigb_ref, bneb_ref, cf2_s)

    @pl.when(s == _S_D)
    def _stage_d():
        y = (_dot(cf0_s[...], wc_ref[0:H, :])
             + _dot(cf1_s[...], wc_ref[H:2 * H, :])
             + _dot(cf2_s[...], wc_ref[2 * H:3 * H, :])
             + bc_ref[...])
        y = jnp.maximum(y, 0.0)
        sem_ref[...] = _dot(y, wsem_ref[...]) + bsem_ref[...]
        feat_ref[...] = jnp.maximum(_dot(y, wc2_ref[...]) + bc2_ref[...], 0.0)


def kernel(parent_feature, Wp, bp, We, be, Wel, bel, Wee, bee, Wne, bne,
           Wc, bc, Wsem, bsem, Wc2, bc2):
    wsem_p = jnp.pad(Wsem, ((0, 0), (0, NSEM_PAD - NSEM)))
    bsem_p = jnp.pad(bsem, (0, NSEM_PAD - NSEM)).reshape(1, NSEM_PAD)
    big_row = jnp.full((1, H), BIG, dtype=jnp.float32)

    def wne_slices(i):
        w4big = jnp.concatenate(
            [jnp.stack([Wne[i, 3 * H + t], big_row[0]]) for t in range(T)], 0)
        return (Wne[i, 0:H], Wne[i, H:2 * H], Wne[i, 2 * H:3 * H],
                w4big, bne[i].reshape(1, H))

    w1a, w2a, w3a, w4biga, bnea = wne_slices(0)
    w1b, w2b, w3b, w4bigb, bneb = wne_slices(1)

    full = lambda shp: pl.BlockSpec(shp, lambda s: tuple(0 for _ in shp))
    wp_spec = pl.BlockSpec((F, PF_BN), lambda s: (0, jnp.minimum(s, N_A - 1)))
    bp_spec = pl.BlockSpec((1, PF_BN), lambda s: (0, jnp.minimum(s, N_A - 1)))
    eel_spec = pl.BlockSpec(
        (BI * C, T), lambda s: (jnp.clip(s - _S_B2, 0, NBLK - 1), 0))

    cel, eel, feats, sem = pl.pallas_call(
        _body,
        grid=(_NSTEPS,),
        in_specs=[
            full((1, F)), wp_spec, bp_spec,
            full((H, 1)), full((1, 1)), full((H, H)), full((H, H)),
            full((1, H)), full((H, T)), full((1, T)),
            full((H, H)), full((H, H)), full((H, H)), full((2 * T, H)),
            full((1, H)),
            full((H, H)), full((H, H)), full((H, H)), full((2 * T, H)),
            full((1, H)),
            full((3 * H, H)), full((1, H)), full((H, NSEM_PAD)),
            full((1, NSEM_PAD)), full((H, F)), full((1, F)),
        ],
        out_specs=[
            full((C, 1)), eel_spec, full((C, F)), full((C, NSEM_PAD)),
        ],
        out_shape=[
            jax.ShapeDtypeStruct((C, 1), jnp.float32),
            jax.ShapeDtypeStruct((C * C, T), jnp.float32),
            jax.ShapeDtypeStruct((C, F), jnp.float32),
            jax.ShapeDtypeStruct((C, NSEM_PAD), jnp.float32),
        ],
        scratch_shapes=[
            pltpu.VMEM((C, H), jnp.float32),
            pltpu.VMEM((C, H), jnp.float32),
            pltpu.VMEM((C, H), jnp.float32),
            pltpu.VMEM((C, 1), jnp.float32),
            pltpu.VMEM((C * C, H), jnp.float32),
            pltpu.VMEM((C * C, T), jnp.float32),
            pltpu.VMEM((C * C, T), jnp.float32),
            pltpu.VMEM((C, 1), jnp.float32),
            pltpu.VMEM((C, H), jnp.float32),
            pltpu.VMEM((C, H), jnp.float32),
        ],
    )(parent_feature, Wp, bp.reshape(1, H * C), We, be.reshape(1, 1),
      Wel[0:H], Wel[H:2 * H], bel.reshape(1, H), Wee, bee.reshape(1, T),
      w1a, w2a, w3a, w4biga, bnea, w1b, w2b, w3b, w4bigb, bneb,
      Wc, bc.reshape(1, H), wsem_p, bsem_p, Wc2, bc2.reshape(1, F))

    return (feats.reshape(1, C, F),
            sem[:, :NSEM].reshape(1, C, NSEM),
            cel.reshape(1, C, 1),
            eel.reshape(1, C, C, T))


# speedup vs baseline: 44.6862x; 1.0162x over previous
import jax
import jax.numpy as jnp
from jax.experimental import pallas as pl
from jax.experimental.pallas import tpu as pltpu

C = 128
H = 256
F = 256
T = 4
IT = 2
NSEM = 57
NSEM_PAD = 64

BI = 32
NBLK = C // BI
PF_BN = 4096
PF_ROWS = PF_BN // H
N_A = (H * C) // PF_BN
BIG = 1e30

_S_A = 0
_S_B1 = _S_A + N_A
_S_B2 = _S_B1 + 1
_S_MP0 = _S_B2 + NBLK
_S_MP1 = _S_MP0 + NBLK
_S_D = _S_MP1 + NBLK
_NSTEPS = _S_D + 1


def _dot(a, b):
    return jnp.dot(a, b, preferred_element_type=jnp.float32)


def _mp_stage(i, cf_s, el_s, coeff_s, cnt_s,
              w1_ref, w2_ref, w3_ref, w4big_ref, bne_ref, out_s):
    i0 = i * BI
    cf = cf_s[...]
    a_rows = cf_s[pl.ds(i0, BI), :]
    a_msg = _dot(a_rows, w1_ref[...])
    b_msg = _dot(cf, w2_ref[...]) + bne_ref[...]
    el = el_s[pl.ds(i0 * C, BI * C), :]
    elw = _dot(el, w3_ref[...]).reshape(BI, C, H)
    base = (elw + a_msg[:, None, :] + b_msg[None, :, :]).reshape(BI * C, H)
    coeff = coeff_s[pl.ds(i0 * C, BI * C), :]
    mfm1 = jnp.where(coeff > 0, 0.0, -1.0)
    nt = jnp.zeros((BI * C, H), dtype=jnp.float32)
    for t in range(T):
        lhs = jnp.concatenate([coeff[:, t:t + 1], mfm1[:, t:t + 1]], axis=1)
        z_t = _dot(lhs, w4big_ref[pl.ds(2 * t, 2), :])
        nt = nt + jnp.maximum(base + z_t, 0.0)
    rows = jax.lax.broadcasted_iota(jnp.int32, (BI, BI * C), 0)
    cols = jax.lax.broadcasted_iota(jnp.int32, (BI, BI * C), 1)
    sel = (cols // C == rows).astype(jnp.float32)
    sums = _dot(sel, nt)
    denom = jnp.maximum(cnt_s[pl.ds(i0, BI), :], 1.0)
    cf_new = sums / denom
    total = jnp.sum(cnt_s[...])
    out_s[pl.ds(i0, BI), :] = jnp.where(total > 0.0, cf_new, a_rows)


def _body(parent_ref, wp_ref, bp_ref, we_ref, be_ref, wela_ref, welb_ref,
          bel_ref, wee_ref, bee_ref,
          w1a_ref, w2a_ref, w3a_ref, w4biga_ref, bnea_ref,
          w1b_ref, w2b_ref, w3b_ref, w4bigb_ref, bneb_ref,
          wc_ref, bc_ref, wsem_ref, bsem_ref, wc2_ref, bc2_ref,
          cel_ref, eel_ref, feat_ref, sem_ref,
          cf0_s, u_s, v_s, cel_s, el_s, coeff_s, cnt_s, cf1_s, cf2_s):
    s = pl.program_id(0)

    @pl.when(s < _S_B1)
    def _stage_a():
        o = jnp.maximum(_dot(parent_ref[...], wp_ref[...]) + bp_ref[...], 0.0)
        cf0_s[pl.ds(s * PF_ROWS, PF_ROWS), :] = o.reshape(PF_ROWS, H)

    @pl.when(s == _S_B1)
    def _stage_b1():
        cf = cf0_s[...]
        cel = _dot(cf, we_ref[...]) + be_ref[...]
        cel_s[...] = cel
        cel_ref[...] = cel
        u_s[...] = _dot(cf, wela_ref[...])
        v_s[...] = _dot(cf, welb_ref[...]) + bel_ref[...]

    @pl.when((s >= _S_B2) & (s < _S_MP0))
    def _stage_b2():
        i = s - _S_B2
        i0 = i * BI
        u_b = u_s[pl.ds(i0, BI), :]
        el = jnp.maximum(u_b[:, None, :] + v_s[...][None, :, :],
                         0.0).reshape(BI * C, H)
        el_s[pl.ds(i0 * C, BI * C), :] = el
        eel = _dot(el, wee_ref[...]) + bee_ref[...]
        eel_ref[...] = eel
        cel = cel_s[...]
        celj = jnp.broadcast_to(cel.reshape(1, C, 1),
                                (BI, C, 1)).reshape(BI * C, 1)
        celi = jnp.broadcast_to(cel_s[pl.ds(i0, BI), :][:, None, :],
                                (BI, C, 1)).reshape(BI * C, 1)
        mask = (eel > 0) & (celi > 0) & (celj > 0)
        mf = mask.astype(jnp.float32)
        coeff_s[pl.ds(i0 * C, BI * C), :] = mf * eel
        s1 = jnp.sum(mf.reshape(BI, C, T), axis=2)
        cnt_s[pl.ds(i0, BI), :] = jnp.sum(s1, axis=1)[:, None]

    @pl.when((s >= _S_MP0) & (s < _S_MP1))
    def _stage_mp0():
        _mp_stage(s - _S_MP0, cf0_s, el_s, coeff_s, cnt_s,
                  w1a_ref, w2a_ref, w3a_ref, w4biga_ref, bnea_ref, cf1_s)

    @pl.when((s >= _S_MP1) & (s < _S_D))
    def _stage_mp1():
        _mp_stage(s - _S_MP1, cf1_s, el_s, coeff_s, cnt_s,
                  w1b_ref, w2b_ref, w3b_ref, w4bigb_ref, bneb_ref, cf2_s)

    @pl.when(s == _S_D)
    def _stage_d():
        y = (_dot(cf0_s[...], wc_ref[0:H, :])
             + _dot(cf1_s[...], wc_ref[H:2 * H, :])
             + _dot(cf2_s[...], wc_ref[2 * H:3 * H, :])
             + bc_ref[...])
        y = jnp.maximum(y, 0.0)
        sem_ref[...] = _dot(y, wsem_ref[...]) + bsem_ref[...]
        feat_ref[...] = jnp.maximum(_dot(y, wc2_ref[...]) + bc2_ref[...], 0.0)


def kernel(parent_feature, Wp, bp, We, be, Wel, bel, Wee, bee, Wne, bne,
           Wc, bc, Wsem, bsem, Wc2, bc2):
    wsem_p = jnp.pad(Wsem, ((0, 0), (0, NSEM_PAD - NSEM)))
    bsem_p = jnp.pad(bsem, (0, NSEM_PAD - NSEM)).reshape(1, NSEM_PAD)
    big_row = jnp.full((1, H), BIG, dtype=jnp.float32)

    def wne_slices(i):
        w4big = jnp.concatenate(
            [jnp.stack([Wne[i, 3 * H + t], big_row[0]]) for t in range(T)], 0)
        return (Wne[i, 0:H], Wne[i, H:2 * H], Wne[i, 2 * H:3 * H],
                w4big, bne[i].reshape(1, H))

    w1a, w2a, w3a, w4biga, bnea = wne_slices(0)
    w1b, w2b, w3b, w4bigb, bneb = wne_slices(1)

    full = lambda shp: pl.BlockSpec(shp, lambda s: tuple(0 for _ in shp))
    wp_spec = pl.BlockSpec((F, PF_BN), lambda s: (0, jnp.minimum(s, N_A - 1)))
    bp_spec = pl.BlockSpec((1, PF_BN), lambda s: (0, jnp.minimum(s, N_A - 1)))
    eel_spec = pl.BlockSpec(
        (BI * C, T), lambda s: (jnp.clip(s - _S_B2, 0, NBLK - 1), 0))

    cel, eel, feats, sem = pl.pallas_call(
        _body,
        grid=(_NSTEPS,),
        in_specs=[
            full((1, F)), wp_spec, bp_spec,
            full((H, 1)), full((1, 1)), full((H, H)), full((H, H)),
            full((1, H)), full((H, T)), full((1, T)),
            full((H, H)), full((H, H)), full((H, H)), full((2 * T, H)),
            full((1, H)),
            full((H, H)), full((H, H)), full((H, H)), full((2 * T, H)),
            full((1, H)),
            full((3 * H, H)), full((1, H)), full((H, NSEM_PAD)),
            full((1, NSEM_PAD)), full((H, F)), full((1, F)),
        ],
        out_specs=[
            full((C, 1)), eel_spec, full((C, F)), full((C, NSEM_PAD)),
        ],
        out_shape=[
            jax.ShapeDtypeStruct((C, 1), jnp.float32),
            jax.ShapeDtypeStruct((C * C, T), jnp.float32),
            jax.ShapeDtypeStruct((C, F), jnp.float32),
            jax.ShapeDtypeStruct((C, NSEM_PAD), jnp.float32),
        ],
        scratch_shapes=[
            pltpu.VMEM((C, H), jnp.float32),
            pltpu.VMEM((C, H), jnp.float32),
            pltpu.VMEM((C, H), jnp.float32),
            pltpu.VMEM((C, 1), jnp.float32),
            pltpu.VMEM((C * C, H), jnp.float32),
            pltpu.VMEM((C * C, T), jnp.float32),
            pltpu.VMEM((C, 1), jnp.float32),
            pltpu.VMEM((C, H), jnp.float32),
            pltpu.VMEM((C, H), jnp.float32),
        ],
    )(parent_feature, Wp, bp.reshape(1, H * C), We, be.reshape(1, 1),
      Wel[0:H], Wel[H:2 * H], bel.reshape(1, H), Wee, bee.reshape(1, T),
      w1a, w2a, w3a, w4biga, bnea, w1b, w2b, w3b, w4bigb, bneb,
      Wc, bc.reshape(1, H), wsem_p, bsem_p, Wc2, bc2.reshape(1, F))

    return (feats.reshape(1, C, F),
            sem[:, :NSEM].reshape(1, C, NSEM),
            cel.reshape(1, C, 1),
            eel.reshape(1, C, C, T))
